# Initial kernel scaffold; baseline (speedup 1.0000x reference)
import functools

import jax
import jax.numpy as jnp
from jax import lax
from jax.experimental import pallas as pl
from jax.experimental.pallas import tpu as pltpu
from jax.experimental.pallas import tpu_sc as plsc

N = 10000
E = 320000
D = 128

NC = 2
NS = 16
NW = NC * NS

NPAD = 10240
RPW = NPAD // NS
CH = 128
EPW = 10112
EPAD = EPW * NW
NCHUNK = EPW // CH

ROW_BLK = 2560


def _sc_agg_body(with_deg, y_hbm, src_hbm, dst_hbm, z2d_hbm, z1d_hbm,
                 parts_hbm, deg_hbm, acc, idx_src, idx_dst, rows, deg_l, sem):
    c = lax.axis_index("c")
    s = lax.axis_index("s")
    wid = s * NC + c

    pltpu.sync_copy(z2d_hbm.at[pl.ds(s * RPW, RPW)], acc.at[pl.ds(s * RPW, RPW)])
    if with_deg:
        pltpu.sync_copy(z1d_hbm.at[:], deg_l)
    plsc.subcore_barrier()

    base = wid * EPW
    ones = jnp.full((16,), 1.0, jnp.float32)

    def chunk(g, carry):
        off = pl.multiple_of(base + g * CH, CH)
        pltpu.sync_copy(src_hbm.at[pl.ds(off, CH)], idx_src)
        pltpu.sync_copy(dst_hbm.at[pl.ds(off, CH)], idx_dst)
        pltpu.async_copy(y_hbm.at[idx_src], rows, sem).wait()
        pltpu.sync_copy(rows, acc.at[idx_dst], add=True)
        if with_deg:
            for j in range(CH // 16):
                idx16 = idx_dst[pl.ds(j * 16, 16)]
                plsc.addupdate_scatter(deg_l, plsc.Indices(idx16), ones)
        return carry

    lax.fori_loop(0, NCHUNK, chunk, 0)
    plsc.subcore_barrier()

    pltpu.sync_copy(acc.at[pl.ds(s * RPW, RPW)],
                    parts_hbm.at[c, pl.ds(s * RPW, RPW)])
    if with_deg:
        pltpu.sync_copy(deg_l, deg_hbm.at[wid])


def _make_sc_agg(with_deg):
    mesh = plsc.VectorSubcoreMesh(core_axis_name="c", subcore_axis_name="s")
    out_type = [jax.ShapeDtypeStruct((NC, NPAD, D), jnp.float32)]
    if with_deg:
        out_type.append(jax.ShapeDtypeStruct((NW, NPAD), jnp.float32))
    scratch = [
        pltpu.VMEM_SHARED((NPAD, D), jnp.float32),
        pltpu.VMEM((CH,), jnp.int32),
        pltpu.VMEM((CH,), jnp.int32),
        pltpu.VMEM((CH, D), jnp.float32),
        pltpu.VMEM((NPAD,), jnp.float32),
        pltpu.SemaphoreType.DMA,
    ]

    def body(y_hbm, src_hbm, dst_hbm, z2d_hbm, z1d_hbm, *rest):
        if with_deg:
            parts_hbm, deg_hbm = rest[0], rest[1]
            scr = rest[2:]
        else:
            parts_hbm, deg_hbm = rest[0], None
            scr = rest[1:]
        _sc_agg_body(with_deg, y_hbm, src_hbm, dst_hbm, z2d_hbm, z1d_hbm,
                     parts_hbm, deg_hbm, *scr)

    return pl.kernel(body, out_type=out_type, mesh=mesh, scratch_types=scratch,
                     name="sc_agg_deg" if with_deg else "sc_agg")


_sc_agg_deg = _make_sc_agg(True)
_sc_agg = _make_sc_agg(False)


def _tc_transform_body(x_ref, w_ref, b_ref, y_ref, z_ref):
    out = jnp.dot(x_ref[...], w_ref[...],
                  preferred_element_type=jnp.float32,
                  precision=lax.Precision.HIGHEST)
    y_ref[...] = out[:, :D]
    z_ref[...] = out[:, D:] + b_ref[0:1, :]


_tc_transform = pl.pallas_call(
    _tc_transform_body,
    grid=(NPAD // ROW_BLK,),
    in_specs=[
        pl.BlockSpec((ROW_BLK, D), lambda i: (i, 0)),
        pl.BlockSpec((D, 2 * D), lambda i: (0, 0)),
        pl.BlockSpec((8, D), lambda i: (0, 0)),
    ],
    out_specs=[
        pl.BlockSpec((ROW_BLK, D), lambda i: (i, 0)),
        pl.BlockSpec((ROW_BLK, D), lambda i: (i, 0)),
    ],
    out_shape=[
        jax.ShapeDtypeStruct((NPAD, D), jnp.float32),
        jax.ShapeDtypeStruct((NPAD, D), jnp.float32),
    ],
)


def _tc_combine_body(p_ref, degp_ref, z1_ref, w_ref, b_ref, y_ref, z_ref):
    agg = p_ref[0] + p_ref[1]
    deg = jnp.maximum(jnp.sum(degp_ref[...], axis=0), 1.0)
    h = jnp.maximum(agg / deg[:, None] + z1_ref[...], 0.0)
    out = jnp.dot(h, w_ref[...],
                  preferred_element_type=jnp.float32,
                  precision=lax.Precision.HIGHEST)
    y_ref[...] = out[:, :D]
    z_ref[...] = out[:, D:] + b_ref[0:1, :]


_tc_combine = pl.pallas_call(
    _tc_combine_body,
    grid=(NPAD // ROW_BLK,),
    in_specs=[
        pl.BlockSpec((NC, ROW_BLK, D), lambda i: (0, i, 0)),
        pl.BlockSpec((NW, ROW_BLK), lambda i: (0, i)),
        pl.BlockSpec((ROW_BLK, D), lambda i: (i, 0)),
        pl.BlockSpec((D, 2 * D), lambda i: (0, 0)),
        pl.BlockSpec((8, D), lambda i: (0, 0)),
    ],
    out_specs=[
        pl.BlockSpec((ROW_BLK, D), lambda i: (i, 0)),
        pl.BlockSpec((ROW_BLK, D), lambda i: (i, 0)),
    ],
    out_shape=[
        jax.ShapeDtypeStruct((NPAD, D), jnp.float32),
        jax.ShapeDtypeStruct((NPAD, D), jnp.float32),
    ],
)


def _tc_final_body(p_ref, degp_ref, z2_ref, out_ref):
    agg = p_ref[0] + p_ref[1]
    deg = jnp.maximum(jnp.sum(degp_ref[...], axis=0), 1.0)
    out_ref[...] = agg / deg[:, None] + z2_ref[...]


_tc_final = pl.pallas_call(
    _tc_final_body,
    grid=(NPAD // ROW_BLK,),
    in_specs=[
        pl.BlockSpec((NC, ROW_BLK, D), lambda i: (0, i, 0)),
        pl.BlockSpec((NW, ROW_BLK), lambda i: (0, i)),
        pl.BlockSpec((ROW_BLK, D), lambda i: (i, 0)),
    ],
    out_specs=pl.BlockSpec((ROW_BLK, D), lambda i: (i, 0)),
    out_shape=jax.ShapeDtypeStruct((NPAD, D), jnp.float32),
)


@jax.jit
def kernel(x, edge_index, W1_l, b1, W1_r, W2_l, b2, W2_r):
    src = edge_index[0].astype(jnp.int32)
    dst = edge_index[1].astype(jnp.int32)
    src = jnp.concatenate([src, jnp.zeros((EPAD - E,), jnp.int32)])
    dst = jnp.concatenate([dst, jnp.full((EPAD - E,), NPAD - 1, jnp.int32)])

    x_pad = jnp.pad(x, ((0, NPAD - N), (0, 0)))
    w1 = jnp.concatenate([W1_l, W1_r], axis=1)
    w2 = jnp.concatenate([W2_l, W2_r], axis=1)
    b1b = jnp.broadcast_to(b1.reshape(1, D), (8, D))
    b2b = jnp.broadcast_to(b2.reshape(1, D), (8, D))
    z2d = jnp.zeros((NPAD, D), jnp.float32)
    z1d = jnp.zeros((NPAD,), jnp.float32)

    y1, z1 = _tc_transform(x_pad, w1, b1b)
    parts1, degp = _sc_agg_deg(y1, src, dst, z2d, z1d)
    y2, z2 = _tc_combine(parts1, degp, z1, w2, b2b)
    parts2 = _sc_agg(y2, src, dst, z2d, z1d)
    out = _tc_final(parts2, degp, z2)
    return out[:N]


# speedup vs baseline: 4.1188x; 4.1188x over previous
import functools

import jax
import jax.numpy as jnp
from jax import lax
from jax.experimental import pallas as pl
from jax.experimental.pallas import tpu as pltpu
from jax.experimental.pallas import tpu_sc as plsc

N = 10000
E = 320000
D = 128

NC = 2
NS = 16
NW = NC * NS

NPAD = 10240
RPW = NPAD // NS
CH = 128
EPW = 10112
EPAD = EPW * NW
NCHUNK = EPW // CH

ROW_BLK = 2560


def _sc_agg_body(with_deg, y_hbm, src_hbm, dst_hbm, z2d_hbm, z1d_hbm,
                 parts_hbm, deg_hbm, acc, idx_src, idx_dst, rows, deg_l, sem):
    c = lax.axis_index("c")
    s = lax.axis_index("s")
    wid = s * NC + c

    pltpu.sync_copy(z2d_hbm.at[pl.ds(s * RPW, RPW)], acc.at[pl.ds(s * RPW, RPW)])
    if with_deg:
        pltpu.sync_copy(z1d_hbm.at[:], deg_l)
    plsc.subcore_barrier()

    base = wid * EPW
    ones = jnp.full((16,), 1.0, jnp.float32)

    def chunk(g, carry):
        off = pl.multiple_of(base + g * CH, CH)
        pltpu.sync_copy(src_hbm.at[pl.ds(off, CH)], idx_src)
        pltpu.sync_copy(dst_hbm.at[pl.ds(off, CH)], idx_dst)
        pltpu.async_copy(y_hbm.at[idx_src], rows, sem).wait()
        pltpu.sync_copy(rows, acc.at[idx_dst], add=True)
        if with_deg:
            for j in range(CH // 16):
                idx16 = idx_dst[pl.ds(j * 16, 16)]
                plsc.addupdate_scatter(deg_l, [idx16], ones)
        return carry

    lax.fori_loop(0, NCHUNK, chunk, 0)
    plsc.subcore_barrier()

    pltpu.sync_copy(acc.at[pl.ds(s * RPW, RPW)],
                    parts_hbm.at[c, pl.ds(s * RPW, RPW)])
    if with_deg:
        pltpu.sync_copy(deg_l, deg_hbm.at[wid])


def _make_sc_agg(with_deg):
    mesh = plsc.VectorSubcoreMesh(core_axis_name="c", subcore_axis_name="s",
                                  num_cores=NC, num_subcores=NS)
    out_type = [jax.ShapeDtypeStruct((NC, NPAD, D), jnp.float32)]
    if with_deg:
        out_type.append(jax.ShapeDtypeStruct((NW, NPAD), jnp.float32))
    scratch = [
        pltpu.VMEM_SHARED((NPAD, D), jnp.float32),
        pltpu.VMEM((CH,), jnp.int32),
        pltpu.VMEM((CH,), jnp.int32),
        pltpu.VMEM((CH, D), jnp.float32),
        pltpu.VMEM((NPAD,), jnp.float32),
        pltpu.SemaphoreType.DMA,
    ]

    def body(y_hbm, src_hbm, dst_hbm, z2d_hbm, z1d_hbm, *rest):
        if with_deg:
            parts_hbm, deg_hbm = rest[0], rest[1]
            scr = rest[2:]
        else:
            parts_hbm, deg_hbm = rest[0], None
            scr = rest[1:]
        _sc_agg_body(with_deg, y_hbm, src_hbm, dst_hbm, z2d_hbm, z1d_hbm,
                     parts_hbm, deg_hbm, *scr)

    return pl.kernel(body, out_type=out_type, mesh=mesh, scratch_types=scratch,
                     compiler_params=pltpu.CompilerParams(
                         needs_layout_passes=False),
                     name="sc_agg_deg" if with_deg else "sc_agg")


_make_sc_agg = functools.lru_cache(maxsize=None)(_make_sc_agg)


def _tc_transform_body(x_ref, w_ref, b_ref, y_ref, z_ref):
    out = jnp.dot(x_ref[...], w_ref[...],
                  preferred_element_type=jnp.float32,
                  precision=lax.Precision.HIGHEST)
    y_ref[...] = out[:, :D]
    z_ref[...] = out[:, D:] + b_ref[0:1, :]


_tc_transform = pl.pallas_call(
    _tc_transform_body,
    grid=(NPAD // ROW_BLK,),
    in_specs=[
        pl.BlockSpec((ROW_BLK, D), lambda i: (i, 0)),
        pl.BlockSpec((D, 2 * D), lambda i: (0, 0)),
        pl.BlockSpec((8, D), lambda i: (0, 0)),
    ],
    out_specs=[
        pl.BlockSpec((ROW_BLK, D), lambda i: (i, 0)),
        pl.BlockSpec((ROW_BLK, D), lambda i: (i, 0)),
    ],
    out_shape=[
        jax.ShapeDtypeStruct((NPAD, D), jnp.float32),
        jax.ShapeDtypeStruct((NPAD, D), jnp.float32),
    ],
)


def _tc_combine_body(p_ref, degp_ref, z1_ref, w_ref, b_ref, y_ref, z_ref):
    agg = p_ref[0] + p_ref[1]
    deg = jnp.maximum(jnp.sum(degp_ref[...], axis=0), 1.0)
    h = jnp.maximum(agg / deg[:, None] + z1_ref[...], 0.0)
    out = jnp.dot(h, w_ref[...],
                  preferred_element_type=jnp.float32,
                  precision=lax.Precision.HIGHEST)
    y_ref[...] = out[:, :D]
    z_ref[...] = out[:, D:] + b_ref[0:1, :]


_tc_combine = pl.pallas_call(
    _tc_combine_body,
    grid=(NPAD // ROW_BLK,),
    in_specs=[
        pl.BlockSpec((NC, ROW_BLK, D), lambda i: (0, i, 0)),
        pl.BlockSpec((NW, ROW_BLK), lambda i: (0, i)),
        pl.BlockSpec((ROW_BLK, D), lambda i: (i, 0)),
        pl.BlockSpec((D, 2 * D), lambda i: (0, 0)),
        pl.BlockSpec((8, D), lambda i: (0, 0)),
    ],
    out_specs=[
        pl.BlockSpec((ROW_BLK, D), lambda i: (i, 0)),
        pl.BlockSpec((ROW_BLK, D), lambda i: (i, 0)),
    ],
    out_shape=[
        jax.ShapeDtypeStruct((NPAD, D), jnp.float32),
        jax.ShapeDtypeStruct((NPAD, D), jnp.float32),
    ],
)


def _tc_final_body(p_ref, degp_ref, z2_ref, out_ref):
    agg = p_ref[0] + p_ref[1]
    deg = jnp.maximum(jnp.sum(degp_ref[...], axis=0), 1.0)
    out_ref[...] = agg / deg[:, None] + z2_ref[...]


_tc_final = pl.pallas_call(
    _tc_final_body,
    grid=(NPAD // ROW_BLK,),
    in_specs=[
        pl.BlockSpec((NC, ROW_BLK, D), lambda i: (0, i, 0)),
        pl.BlockSpec((NW, ROW_BLK), lambda i: (0, i)),
        pl.BlockSpec((ROW_BLK, D), lambda i: (i, 0)),
    ],
    out_specs=pl.BlockSpec((ROW_BLK, D), lambda i: (i, 0)),
    out_shape=jax.ShapeDtypeStruct((NPAD, D), jnp.float32),
)


@jax.jit
def kernel(x, edge_index, W1_l, b1, W1_r, W2_l, b2, W2_r):
    src = edge_index[0].astype(jnp.int32)
    dst = edge_index[1].astype(jnp.int32)
    src = jnp.concatenate([src, jnp.zeros((EPAD - E,), jnp.int32)])
    dst = jnp.concatenate([dst, jnp.full((EPAD - E,), NPAD - 1, jnp.int32)])

    x_pad = jnp.pad(x, ((0, NPAD - N), (0, 0)))
    w1 = jnp.concatenate([W1_l, W1_r], axis=1)
    w2 = jnp.concatenate([W2_l, W2_r], axis=1)
    b1b = jnp.broadcast_to(b1.reshape(1, D), (8, D))
    b2b = jnp.broadcast_to(b2.reshape(1, D), (8, D))
    z2d = jnp.zeros((NPAD, D), jnp.float32)
    z1d = jnp.zeros((NPAD,), jnp.float32)

    y1, z1 = _tc_transform(x_pad, w1, b1b)
    parts1, degp = _make_sc_agg(True)(y1, src, dst, z2d, z1d)
    y2, z2 = _tc_combine(parts1, degp, z1, w2, b2b)
    parts2, = _make_sc_agg(False)(y2, src, dst, z2d, z1d)
    out = _tc_final(parts2, degp, z2)
    return out[:N]
